# baseline (speedup 1.0000x reference)
import functools

import jax
import jax.numpy as jnp
from jax import lax
from jax.experimental import pallas as pl
from jax.experimental.pallas import tpu as pltpu
from jax.experimental.pallas import tpu_sc as plsc

N_BUS = 1000
BATCH = 10
N = N_BUS * BATCH
E = 320000
FP = 16
EPS = 1e-5

NC = 2
NS = 16
NW = NC * NS
CH = 128
NSTEP = 80
EPAD = NW * NSTEP * CH
DPAD = 10016
NB = 8
NGRP = NSTEP // NB
NPAD = 10240
RPW = NPAD // NS
PR = NPAD // 8
RR = N // 8


@functools.cache
def _make_segsum_sc():
    mesh = plsc.VectorSubcoreMesh(core_axis_name="c", subcore_axis_name="s")
    return functools.partial(
        pl.kernel,
        mesh=mesh,
        compiler_params=pltpu.CompilerParams(use_tc_tiling_on_sc=False),
        out_type=jax.ShapeDtypeStruct((NC, NPAD, FP), jnp.float32),
        scratch_types=[
            pltpu.VMEM((NSTEP, CH), jnp.int32),
            pltpu.VMEM((NSTEP, CH), jnp.int32),
            [pltpu.VMEM((CH, FP), jnp.float32)] * NB,
            pltpu.VMEM_SHARED((NPAD, FP), jnp.float32),
            [pltpu.SemaphoreType.DMA] * NB,
            [pltpu.SemaphoreType.DMA] * NB,
        ],
    )(_segsum_body)


def _segsum_body(m_hbm, src_hbm, dst_hbm, zeros_hbm, out_hbm,
                 src_v, dst_v, rows, acc_sh, gsem, ssem):
    c = lax.axis_index("c")
    s = lax.axis_index("s")
    wid = c * NS + s

    pltpu.sync_copy(src_hbm.at[wid], src_v)
    pltpu.sync_copy(dst_hbm.at[wid], dst_v)

    for b in range(NB):
        pltpu.async_copy(m_hbm.at[src_v.at[b]], rows[b], gsem[b])
    pltpu.sync_copy(zeros_hbm.at[pl.ds(s * RPW, RPW)],
                    acc_sh.at[pl.ds(s * RPW, RPW)])
    plsc.subcore_barrier()

    def group(g, reissue):
        base = g * NB
        scatters = []
        for b in range(NB):
            j = base + b
            pltpu.make_async_copy(m_hbm.at[src_v.at[j]], rows[b],
                                  gsem[b]).wait()
            scatters.append(pltpu.async_copy(
                rows[b], acc_sh.at[dst_v.at[j]], ssem[b], add=True))
        for b in range(NB):
            scatters[b].wait()
            if reissue:
                pltpu.async_copy(m_hbm.at[src_v.at[base + NB + b]],
                                 rows[b], gsem[b])
        return 0

    lax.fori_loop(0, NGRP - 1, lambda g, _: group(g, True), 0)
    group(NGRP - 1, False)
    plsc.subcore_barrier()

    pltpu.sync_copy(acc_sh.at[pl.ds(s * RPW, RPW)],
                    out_hbm.at[c, pl.ds(s * RPW, RPW)])


_HI = lax.Precision.HIGHEST


def _proj_body(x3_ref, wrel_ref, wroot_ref, m_ref, r_ref):
    mcols = []
    rcols = []
    for a in range(8):
        xa = x3_ref[:, a, :]
        mcols.append(jnp.dot(xa, wrel_ref[...],
                             preferred_element_type=jnp.float32, precision=_HI))
        rcols.append(jnp.dot(xa, wroot_ref[...],
                             preferred_element_type=jnp.float32, precision=_HI))
    m_ref[...] = jnp.concatenate(mcols, axis=1)
    r_ref[...] = jnp.concatenate(rcols, axis=1)


def _bn_relu(h, g, bt, gmat):
    mask = lax.broadcasted_iota(jnp.int32, (PR, 128), 0) < RR
    hm = jnp.where(mask, h, 0.0)
    s1 = jnp.sum(hm, axis=0, keepdims=True)
    s2 = jnp.sum(hm * hm, axis=0, keepdims=True)
    mu = jnp.dot(s1, gmat, preferred_element_type=jnp.float32,
                 precision=_HI) * (1.0 / N)
    ex2 = jnp.dot(s2, gmat, preferred_element_type=jnp.float32,
                  precision=_HI) * (1.0 / N)
    var = ex2 - mu * mu
    return jnp.maximum(g * (h - mu) * lax.rsqrt(var + EPS) + bt, 0.0)


def _mid_body(r_ref, p0_ref, p1_ref, b_ref, g_ref, bt_ref, gmat_ref,
              wrelb_ref, wrootb_ref, m2_ref, r2_ref):
    h = r_ref[...] + p0_ref[...] + p1_ref[...] + b_ref[...]
    h1 = _bn_relu(h, g_ref[...], bt_ref[...], gmat_ref[...])
    m2_ref[...] = jnp.dot(h1, wrelb_ref[...],
                          preferred_element_type=jnp.float32, precision=_HI)
    r2_ref[...] = jnp.dot(h1, wrootb_ref[...],
                          preferred_element_type=jnp.float32, precision=_HI)


def _last_body(r_ref, p0_ref, p1_ref, b_ref, g_ref, bt_ref, gmat_ref, h2_ref):
    h = r_ref[...] + p0_ref[...] + p1_ref[...] + b_ref[...]
    h2_ref[...] = _bn_relu(h, g_ref[...], bt_ref[...], gmat_ref[...])


def _head_body(hf_ref, wl1_ref, bl1_ref, wl2_ref, bl2_ref, out_ref):
    hid = jnp.dot(hf_ref[...], wl1_ref[...],
                  preferred_element_type=jnp.float32, precision=_HI)
    hid = jnp.maximum(hid + bl1_ref[...], 0.0)
    out_ref[...] = (jnp.dot(hid, wl2_ref[...],
                            preferred_element_type=jnp.float32, precision=_HI)
                    + bl2_ref[...])


_f32 = jnp.float32


def _pad16(w):
    return jnp.pad(w, ((0, 0), (0, FP - w.shape[1])))


def _tile_v(v):
    return jnp.tile(jnp.pad(v, (0, FP - v.shape[0])), 8).reshape(1, 128)


def kernel(x, edge_index, W1_root, W1_rel, b1, g1, bt1,
           W2_root, W2_rel, b2, g2, bt2, Wl1, bl1, Wl2, bl2):
    pad_n = EPAD - E
    src3 = jnp.concatenate(
        [edge_index[0], jnp.zeros((pad_n,), jnp.int32)]).reshape(NW, NSTEP, CH)
    dst3 = jnp.concatenate(
        [edge_index[1], jnp.full((pad_n,), DPAD, jnp.int32)]).reshape(NW, NSTEP, CH)
    zeros_pad = jnp.zeros((NPAD, FP), _f32)

    x3 = jnp.pad(x, ((0, NPAD - N), (0, 0))).reshape(PR, 8, 128)
    w1rel = _pad16(W1_rel)
    w1root = _pad16(W1_root)
    eye8 = jnp.eye(8, dtype=_f32)
    w2relb = jnp.kron(eye8, _pad16(jnp.pad(W2_rel, ((0, FP - 12), (0, 0)))))
    w2rootb = jnp.kron(eye8, _pad16(jnp.pad(W2_root, ((0, FP - 12), (0, 0)))))
    gmat = jnp.tile(jnp.eye(FP, dtype=_f32), (8, 8))

    m1, r1 = pl.pallas_call(
        _proj_body,
        out_shape=[jax.ShapeDtypeStruct((PR, 128), _f32)] * 2,
    )(x3, w1rel, w1root)

    segsum = _make_segsum_sc()
    part1 = segsum(m1.reshape(NPAD, FP), src3, dst3, zeros_pad)
    part1p = part1.reshape(NC, PR, 128)

    m2, r2 = pl.pallas_call(
        _mid_body,
        out_shape=[jax.ShapeDtypeStruct((PR, 128), _f32)] * 2,
    )(r1, part1p[0], part1p[1], _tile_v(b1), _tile_v(g1), _tile_v(bt1),
      gmat, w2relb, w2rootb)

    part2 = segsum(m2.reshape(NPAD, FP), src3, dst3, zeros_pad)
    part2p = part2.reshape(NC, PR, 128)

    h2 = pl.pallas_call(
        _last_body,
        out_shape=jax.ShapeDtypeStruct((PR, 128), _f32),
    )(r2, part2p[0], part2p[1], _tile_v(b2), _tile_v(g2), _tile_v(bt2), gmat)

    hf = h2.reshape(NPAD, FP)[:N, :12].reshape(BATCH, N_BUS * 12)

    out = pl.pallas_call(
        _head_body,
        out_shape=jax.ShapeDtypeStruct((BATCH, 2 * N_BUS), _f32),
    )(hf, Wl1, bl1.reshape(1, -1), Wl2, bl2.reshape(1, -1))
    return out


# speedup vs baseline: 1.5668x; 1.5668x over previous
import functools

import jax
import jax.numpy as jnp
from jax import lax
from jax.experimental import pallas as pl
from jax.experimental.pallas import tpu as pltpu
from jax.experimental.pallas import tpu_sc as plsc

N_BUS = 1000
BATCH = 10
N = N_BUS * BATCH
E = 320000
FP = 16
EPS = 1e-5

NC = 2
NS = 16
NW = NC * NS
CH = 128
NSTEP = 80
EPAD = NW * NSTEP * CH
DPAD = 10016
NB = 8
NGRP = NSTEP // NB
NPAD = 10240
RPW = NPAD // NS
PR = NPAD // 8
RR = N // 8


@functools.cache
def _make_segsum_sc():
    mesh = plsc.VectorSubcoreMesh(core_axis_name="c", subcore_axis_name="s")
    return functools.partial(
        pl.kernel,
        mesh=mesh,
        compiler_params=pltpu.CompilerParams(use_tc_tiling_on_sc=False),
        out_type=jax.ShapeDtypeStruct((NC, NPAD, FP), jnp.float32),
        scratch_types=[
            pltpu.VMEM((NSTEP, CH), jnp.int32),
            pltpu.VMEM((NSTEP, CH), jnp.int32),
            [pltpu.VMEM((CH, FP), jnp.float32)] * NB,
            pltpu.VMEM_SHARED((NPAD, FP), jnp.float32),
            pltpu.VMEM_SHARED((NPAD, FP), jnp.float32),
            [pltpu.SemaphoreType.DMA] * NB,
            [pltpu.SemaphoreType.DMA] * NB,
        ],
    )(_segsum_body)


def _segsum_body(m_hbm, src_hbm, dst_hbm, zeros_hbm, out_hbm,
                 src_v, dst_v, rows, acc_sh, table_sh, gsem, ssem):
    c = lax.axis_index("c")
    s = lax.axis_index("s")
    wid = c * NS + s

    pltpu.sync_copy(src_hbm.at[wid], src_v)
    pltpu.sync_copy(dst_hbm.at[wid], dst_v)
    pltpu.sync_copy(m_hbm.at[pl.ds(s * RPW, RPW)],
                    table_sh.at[pl.ds(s * RPW, RPW)])
    pltpu.sync_copy(zeros_hbm.at[pl.ds(s * RPW, RPW)],
                    acc_sh.at[pl.ds(s * RPW, RPW)])
    plsc.subcore_barrier()

    for b in range(NB):
        pltpu.async_copy(table_sh.at[src_v.at[b]], rows[b], gsem[b])

    def group(g, reissue):
        base = g * NB
        scatters = []
        for b in range(NB):
            j = base + b
            pltpu.make_async_copy(table_sh.at[src_v.at[j]], rows[b],
                                  gsem[b]).wait()
            scatters.append(pltpu.async_copy(
                rows[b], acc_sh.at[dst_v.at[j]], ssem[b], add=True))
        for b in range(NB):
            scatters[b].wait()
            if reissue:
                pltpu.async_copy(table_sh.at[src_v.at[base + NB + b]],
                                 rows[b], gsem[b])
        return 0

    lax.fori_loop(0, NGRP - 1, lambda g, _: group(g, True), 0)
    group(NGRP - 1, False)
    plsc.subcore_barrier()

    pltpu.sync_copy(acc_sh.at[pl.ds(s * RPW, RPW)],
                    out_hbm.at[c, pl.ds(s * RPW, RPW)])


_HI = lax.Precision.HIGHEST


def _proj_body(x3_ref, wrel_ref, wroot_ref, m_ref, r_ref):
    mcols = []
    rcols = []
    for a in range(8):
        xa = x3_ref[:, a, :]
        mcols.append(jnp.dot(xa, wrel_ref[...],
                             preferred_element_type=jnp.float32, precision=_HI))
        rcols.append(jnp.dot(xa, wroot_ref[...],
                             preferred_element_type=jnp.float32, precision=_HI))
    m_ref[...] = jnp.concatenate(mcols, axis=1)
    r_ref[...] = jnp.concatenate(rcols, axis=1)


def _bn_relu(h, g, bt, gmat):
    mask = lax.broadcasted_iota(jnp.int32, (PR, 128), 0) < RR
    hm = jnp.where(mask, h, 0.0)
    s1 = jnp.sum(hm, axis=0, keepdims=True)
    s2 = jnp.sum(hm * hm, axis=0, keepdims=True)
    mu = jnp.dot(s1, gmat, preferred_element_type=jnp.float32,
                 precision=_HI) * (1.0 / N)
    ex2 = jnp.dot(s2, gmat, preferred_element_type=jnp.float32,
                  precision=_HI) * (1.0 / N)
    var = ex2 - mu * mu
    return jnp.maximum(g * (h - mu) * lax.rsqrt(var + EPS) + bt, 0.0)


def _mid_body(r_ref, p0_ref, p1_ref, b_ref, g_ref, bt_ref, gmat_ref,
              wrelb_ref, wrootb_ref, m2_ref, r2_ref):
    h = r_ref[...] + p0_ref[...] + p1_ref[...] + b_ref[...]
    h1 = _bn_relu(h, g_ref[...], bt_ref[...], gmat_ref[...])
    m2_ref[...] = jnp.dot(h1, wrelb_ref[...],
                          preferred_element_type=jnp.float32, precision=_HI)
    r2_ref[...] = jnp.dot(h1, wrootb_ref[...],
                          preferred_element_type=jnp.float32, precision=_HI)


def _last_body(r_ref, p0_ref, p1_ref, b_ref, g_ref, bt_ref, gmat_ref, h2_ref):
    h = r_ref[...] + p0_ref[...] + p1_ref[...] + b_ref[...]
    h2_ref[...] = _bn_relu(h, g_ref[...], bt_ref[...], gmat_ref[...])


def _head_body(hf_ref, wl1_ref, bl1_ref, wl2_ref, bl2_ref, out_ref):
    hid = jnp.dot(hf_ref[...], wl1_ref[...],
                  preferred_element_type=jnp.float32, precision=_HI)
    hid = jnp.maximum(hid + bl1_ref[...], 0.0)
    out_ref[...] = (jnp.dot(hid, wl2_ref[...],
                            preferred_element_type=jnp.float32, precision=_HI)
                    + bl2_ref[...])


_f32 = jnp.float32


def _pad16(w):
    return jnp.pad(w, ((0, 0), (0, FP - w.shape[1])))


def _tile_v(v):
    return jnp.tile(jnp.pad(v, (0, FP - v.shape[0])), 8).reshape(1, 128)


def kernel(x, edge_index, W1_root, W1_rel, b1, g1, bt1,
           W2_root, W2_rel, b2, g2, bt2, Wl1, bl1, Wl2, bl2):
    pad_n = EPAD - E
    src3 = jnp.concatenate(
        [edge_index[0], jnp.zeros((pad_n,), jnp.int32)]).reshape(NW, NSTEP, CH)
    dst3 = jnp.concatenate(
        [edge_index[1], jnp.full((pad_n,), DPAD, jnp.int32)]).reshape(NW, NSTEP, CH)
    zeros_pad = jnp.zeros((NPAD, FP), _f32)

    x3 = jnp.pad(x, ((0, NPAD - N), (0, 0))).reshape(PR, 8, 128)
    w1rel = _pad16(W1_rel)
    w1root = _pad16(W1_root)
    eye8 = jnp.eye(8, dtype=_f32)
    w2relb = jnp.kron(eye8, _pad16(jnp.pad(W2_rel, ((0, FP - 12), (0, 0)))))
    w2rootb = jnp.kron(eye8, _pad16(jnp.pad(W2_root, ((0, FP - 12), (0, 0)))))
    gmat = jnp.tile(jnp.eye(FP, dtype=_f32), (8, 8))

    m1, r1 = pl.pallas_call(
        _proj_body,
        out_shape=[jax.ShapeDtypeStruct((PR, 128), _f32)] * 2,
    )(x3, w1rel, w1root)

    segsum = _make_segsum_sc()
    part1 = segsum(m1.reshape(NPAD, FP), src3, dst3, zeros_pad)
    part1p = part1.reshape(NC, PR, 128)

    m2, r2 = pl.pallas_call(
        _mid_body,
        out_shape=[jax.ShapeDtypeStruct((PR, 128), _f32)] * 2,
    )(r1, part1p[0], part1p[1], _tile_v(b1), _tile_v(g1), _tile_v(bt1),
      gmat, w2relb, w2rootb)

    part2 = segsum(m2.reshape(NPAD, FP), src3, dst3, zeros_pad)
    part2p = part2.reshape(NC, PR, 128)

    h2 = pl.pallas_call(
        _last_body,
        out_shape=jax.ShapeDtypeStruct((PR, 128), _f32),
    )(r2, part2p[0], part2p[1], _tile_v(b2), _tile_v(g2), _tile_v(bt2), gmat)

    hf = h2.reshape(NPAD, FP)[:N, :12].reshape(BATCH, N_BUS * 12)

    out = pl.pallas_call(
        _head_body,
        out_shape=jax.ShapeDtypeStruct((BATCH, 2 * N_BUS), _f32),
    )(hf, Wl1, bl1.reshape(1, -1), Wl2, bl2.reshape(1, -1))
    return out
